# baseline (speedup 1.0000x reference)
import jax
import jax.numpy as jnp
from jax import lax
from jax.experimental import pallas as pl
from jax.experimental.pallas import tpu as pltpu
from jax.experimental.pallas import tpu_sc as plsc

_NUM_UNITS = 64
_MAX_REL = 128
_LQ = 2048
_LK = 2048

_NC = 2
_NS = 16
_NW = _NC * _NS

_R = _LQ // _NW
_CK = 1024
_SROWS = _CK + _R - 1


def _band_expand(v_hbm, out_hbm, stage, sem):
  wid = lax.axis_index("s") * _NC + lax.axis_index("c")
  q0 = wid * _R
  for kb in range(_LK // _CK):
    k0 = kb * _CK
    start = (_LK - 1) - (q0 + _R - 1) + k0
    pltpu.sync_copy(v_hbm.at[pl.ds(start, _SROWS)], stage)

    def fire(r, carry):
      src = stage.at[pl.ds(_R - 1 - r, _CK)]
      dst = out_hbm.at[pl.ds((q0 + r) * _LK + k0, _CK)]
      pltpu.async_copy(src, dst, sem)
      return carry

    lax.fori_loop(0, _R, fire, 0)

    def drain(r, carry):
      pltpu.make_async_copy(
          v_hbm.at[pl.ds(0, _CK)], stage.at[pl.ds(0, _CK)], sem
      ).wait()
      return carry

    lax.fori_loop(0, _R, drain, 0)


def kernel(length_q, length_k, embeddings_table):
  t = embeddings_table.astype(jnp.float32)
  v = jnp.concatenate(
      [
          jnp.broadcast_to(t[0:1], (_LK - 1 - _MAX_REL, _NUM_UNITS)),
          t,
          jnp.broadcast_to(t[2 * _MAX_REL:, :], (_LQ - 1 - _MAX_REL, _NUM_UNITS)),
      ],
      axis=0,
  )

  mesh = plsc.VectorSubcoreMesh(
      core_axis_name="c", subcore_axis_name="s", num_cores=_NC, num_subcores=_NS
  )
  out = pl.kernel(
      _band_expand,
      out_type=jax.ShapeDtypeStruct((_LQ * _LK, _NUM_UNITS), jnp.float32),
      mesh=mesh,
      scratch_types=[
          pltpu.VMEM((_SROWS, _NUM_UNITS), jnp.float32),
          pltpu.SemaphoreType.DMA,
      ],
      compiler_params=pltpu.CompilerParams(use_tc_tiling_on_sc=False),
  )(v)
  return out


# speedup vs baseline: 5.6632x; 5.6632x over previous
import jax
import jax.numpy as jnp
from jax import lax
from jax.experimental import pallas as pl
from jax.experimental.pallas import tpu as pltpu
from jax.experimental.pallas import tpu_sc as plsc

_NUM_UNITS = 64
_MAX_REL = 128
_LQ = 2048
_LK = 2048

_NC = 2
_NS = 16
_NW = _NC * _NS

_NCLS = 128
_M = _LQ // _NCLS
_CPW = _NCLS // _NW
_TR = _NUM_UNITS // 8
_TRH = 4
_NTC = _LK // 128
_STC = _NTC + _M - 1
_VLO = 14
_VHI = 17


def _band_expand(vt_hbm, out_hbm, stage, sem_g, sem_s):
  wid = lax.axis_index("s") * _NC + lax.axis_index("c")

  def drain_g(n):
    def body(t, carry):
      pltpu.make_async_copy(
          out_hbm.at[0, pl.ds(0, _TRH), 0], stage.at[:, 0], sem_g
      ).wait()
      return carry

    lax.fori_loop(0, n, body, 0)

  for h in range(2):
    trb = h * _TRH

    def fill_lo(t, carry):
      pltpu.async_copy(
          vt_hbm.at[pl.ds(trb, _TRH), :, pl.ds(0, 128)],
          stage.at[:, t],
          sem_g,
      )
      return carry

    def fill_hi(t, carry):
      pltpu.async_copy(
          vt_hbm.at[pl.ds(trb, _TRH), :, pl.ds(2304, 128)],
          stage.at[:, t],
          sem_g,
      )
      return carry

    lax.fori_loop(0, _VLO, fill_lo, 0)
    lax.fori_loop(_VHI, _STC, fill_hi, 0)
    drain_g(_VLO + (_STC - _VHI))

    for ci in range(_CPW):
      c = wid + _NW * ci
      base = (_NCLS - 1) - c
      p = lax.rem(base, 8)
      aligned = pl.multiple_of(base - p, 8)

      def fire_var(t, carry):
        pltpu.async_copy(
            vt_hbm.at[pl.ds(p * 8 + trb, _TRH), :, pl.ds(aligned + 128 * t, 128)],
            stage.at[:, t],
            sem_g,
        )
        return carry

      lax.fori_loop(_VLO, _VHI, fire_var, 0)
      drain_g(_VHI - _VLO)

      def fire_planes(m, carry):
        q = c + _NCLS * m
        tcx0 = (_M - 1) - m
        pltpu.async_copy(
            stage.at[:, pl.ds(tcx0, _NTC)],
            out_hbm.at[q, pl.ds(trb, _TRH)],
            sem_s,
        )
        return carry

      lax.fori_loop(0, _M, fire_planes, 0)

      def drain_planes(m, carry):
        pltpu.make_async_copy(
            out_hbm.at[0, pl.ds(0, _TRH)], stage.at[:, pl.ds(0, _NTC)], sem_s
        ).wait()
        return carry

      lax.fori_loop(0, _M, drain_planes, 0)


def kernel(length_q, length_k, embeddings_table):
  tt = embeddings_table.astype(jnp.float32).T
  vt = jnp.concatenate(
      [
          jnp.broadcast_to(tt[:, :1], (_NUM_UNITS, _LK - 1 - _MAX_REL)),
          tt,
          jnp.broadcast_to(tt[:, -1:], (_NUM_UNITS, _LQ - _MAX_REL)),
      ],
      axis=1,
  )
  vtp = jnp.concatenate(
      [vt, jnp.broadcast_to(vt[:, -1:], (_NUM_UNITS, 7))], axis=1
  )
  vt8 = jnp.stack([vtp[:, p : p + _LQ + _LK] for p in range(8)])
  vt8 = vt8.reshape(8 * _TR, 8, _LQ + _LK)

  mesh = plsc.VectorSubcoreMesh(
      core_axis_name="c", subcore_axis_name="s", num_cores=_NC, num_subcores=_NS
  )
  out5 = pl.kernel(
      _band_expand,
      out_type=jax.ShapeDtypeStruct((_LQ, _TR, _NTC, 8, 128), jnp.float32),
      mesh=mesh,
      scratch_types=[
          pltpu.VMEM((_TRH, _STC, 8, 128), jnp.float32),
          pltpu.SemaphoreType.DMA,
          pltpu.SemaphoreType.DMA,
      ],
      compiler_params=pltpu.CompilerParams(use_tc_tiling_on_sc=False),
  )(vt8)
  return jnp.transpose(out5, (0, 2, 4, 1, 3)).reshape(_LQ, _LK, _NUM_UNITS)


# speedup vs baseline: 6.8378x; 1.2074x over previous
import jax
import jax.numpy as jnp
from jax import lax
from jax.experimental import pallas as pl
from jax.experimental.pallas import tpu as pltpu
from jax.experimental.pallas import tpu_sc as plsc

_NUM_UNITS = 64
_MAX_REL = 128
_LQ = 2048
_LK = 2048

_NC = 2
_NS = 16
_NW = _NC * _NS

_NCLS = 128
_M = _LQ // _NCLS
_CPW = _NCLS // _NW
_TR = _NUM_UNITS // 8
_TRH = 4
_NTC = _LK // 128
_STC = _NTC + _M - 1
_VLO = 14
_VHI = 17


def _band_expand(vt_hbm, out_hbm, stage, sem_g, sem_s):
  wid = lax.axis_index("s") * _NC + lax.axis_index("c")

  def drain_g(n):
    def body(t, carry):
      pltpu.make_async_copy(
          out_hbm.at[0, pl.ds(0, _TRH), 0], stage.at[:, 0], sem_g
      ).wait()
      return carry

    lax.fori_loop(0, n, body, 0)

  for h in range(2):
    trb = h * _TRH

    def fill_lo(t, carry):
      pltpu.async_copy(
          vt_hbm.at[pl.ds(trb, _TRH), :, pl.ds(128 * t, 128)],
          stage.at[:, t],
          sem_g,
      )
      return carry

    def fill_hi(t, carry):
      pltpu.async_copy(
          vt_hbm.at[pl.ds(trb, _TRH), :, pl.ds(2304 + 128 * (t - _VHI), 128)],
          stage.at[:, t],
          sem_g,
      )
      return carry

    lax.fori_loop(0, _VLO, fill_lo, 0)
    lax.fori_loop(_VHI, _STC, fill_hi, 0)
    drain_g(_VLO + (_STC - _VHI))

    for ci in range(_CPW):
      c = wid + _NW * ci
      base = (_NCLS - 1) - c
      p = lax.rem(base, 8)
      aligned = pl.multiple_of(base - p, 8)

      def fire_var(t, carry):
        pltpu.async_copy(
            vt_hbm.at[pl.ds(p * 8 + trb, _TRH), :, pl.ds(aligned + 128 * t, 128)],
            stage.at[:, t],
            sem_g,
        )
        return carry

      lax.fori_loop(_VLO, _VHI, fire_var, 0)
      drain_g(_VHI - _VLO)

      def fire_planes(m, carry):
        q = c + _NCLS * m
        tcx0 = (_M - 1) - m
        pltpu.async_copy(
            stage.at[:, pl.ds(tcx0, _NTC)],
            out_hbm.at[q, pl.ds(trb, _TRH)],
            sem_s,
        )
        return carry

      lax.fori_loop(0, _M, fire_planes, 0)

      def drain_planes(m, carry):
        pltpu.make_async_copy(
            out_hbm.at[0, pl.ds(0, _TRH)], stage.at[:, pl.ds(0, _NTC)], sem_s
        ).wait()
        return carry

      lax.fori_loop(0, _M, drain_planes, 0)


def kernel(length_q, length_k, embeddings_table):
  tt = embeddings_table.astype(jnp.float32).T
  vt = jnp.concatenate(
      [
          jnp.broadcast_to(tt[:, :1], (_NUM_UNITS, _LK - 1 - _MAX_REL)),
          tt,
          jnp.broadcast_to(tt[:, -1:], (_NUM_UNITS, _LQ - _MAX_REL)),
      ],
      axis=1,
  )
  vtp = jnp.concatenate(
      [vt, jnp.broadcast_to(vt[:, -1:], (_NUM_UNITS, 7))], axis=1
  )
  vt8 = jnp.stack([vtp[:, p : p + _LQ + _LK] for p in range(8)])
  vt8 = vt8.reshape(8 * _TR, 8, _LQ + _LK)

  mesh = plsc.VectorSubcoreMesh(
      core_axis_name="c", subcore_axis_name="s", num_cores=_NC, num_subcores=_NS
  )
  out5 = pl.kernel(
      _band_expand,
      out_type=jax.ShapeDtypeStruct((_LQ, _TR, _NTC, 8, 128), jnp.float32),
      mesh=mesh,
      scratch_types=[
          pltpu.VMEM((_TRH, _STC, 8, 128), jnp.float32),
          pltpu.SemaphoreType.DMA,
          pltpu.SemaphoreType.DMA,
      ],
      compiler_params=pltpu.CompilerParams(use_tc_tiling_on_sc=False),
  )(vt8)
  return jnp.transpose(out5, (0, 2, 4, 1, 3)).reshape(_LQ, _LK, _NUM_UNITS)


# speedup vs baseline: 6.8451x; 1.0011x over previous
import jax
import jax.numpy as jnp
from jax import lax
from jax.experimental import pallas as pl
from jax.experimental.pallas import tpu as pltpu
from jax.experimental.pallas import tpu_sc as plsc

_NUM_UNITS = 64
_MAX_REL = 128
_LQ = 2048
_LK = 2048

_NC = 2
_NS = 16
_NW = _NC * _NS

_NCLS = 128
_M = _LQ // _NCLS
_CPW = _NCLS // _NW
_TR = _NUM_UNITS // 8
_TRH = 4
_NTC = _LK // 128
_STC = _NTC + _M - 1
_VLO = 14
_VHI = 17


def _band_expand(vt_hbm, out_hbm, stage, sem_g, sem_s):
  wid = lax.axis_index("s") * _NC + lax.axis_index("c")
  jit8 = pl.multiple_of(8 * wid, 8)

  def drain_g(n):
    def body(t, carry):
      pltpu.make_async_copy(
          out_hbm.at[0, pl.ds(0, _TRH), 0], stage.at[:, 0], sem_g
      ).wait()
      return carry

    lax.fori_loop(0, n, body, 0)

  for h in range(2):
    trb = h * _TRH

    def fill_lo(t, carry):
      pltpu.async_copy(
          vt_hbm.at[pl.ds(trb, _TRH), :, pl.ds(jit8 + 112 * t, 128)],
          stage.at[:, t],
          sem_g,
      )
      return carry

    def fill_hi(t, carry):
      pltpu.async_copy(
          vt_hbm.at[pl.ds(trb, _TRH), :, pl.ds(2176 + jit8 + 112 * (t - _VHI), 128)],
          stage.at[:, t],
          sem_g,
      )
      return carry

    lax.fori_loop(0, _VLO, fill_lo, 0)
    lax.fori_loop(_VHI, _STC, fill_hi, 0)
    drain_g(_VLO + (_STC - _VHI))

    for ci in range(_CPW):
      c = wid + _NW * ci
      base = (_NCLS - 1) - c
      p = lax.rem(base, 8)
      aligned = pl.multiple_of(base - p, 8)

      def fire_var(t, carry):
        pltpu.async_copy(
            vt_hbm.at[pl.ds(p * 8 + trb, _TRH), :, pl.ds(aligned + 128 * t, 128)],
            stage.at[:, t],
            sem_g,
        )
        return carry

      lax.fori_loop(_VLO, _VHI, fire_var, 0)
      drain_g(_VHI - _VLO)

      def fire_planes(m, carry):
        q = c + _NCLS * m
        tcx0 = (_M - 1) - m
        pltpu.async_copy(
            stage.at[:, pl.ds(tcx0, _NTC)],
            out_hbm.at[q, pl.ds(trb, _TRH)],
            sem_s,
        )
        return carry

      lax.fori_loop(0, _M, fire_planes, 0)

      def drain_planes(m, carry):
        pltpu.make_async_copy(
            out_hbm.at[0, pl.ds(0, _TRH)], stage.at[:, pl.ds(0, _NTC)], sem_s
        ).wait()
        return carry

      lax.fori_loop(0, _M, drain_planes, 0)


def kernel(length_q, length_k, embeddings_table):
  tt = embeddings_table.astype(jnp.float32).T
  vt = jnp.concatenate(
      [
          jnp.broadcast_to(tt[:, :1], (_NUM_UNITS, _LK - 1 - _MAX_REL)),
          tt,
          jnp.broadcast_to(tt[:, -1:], (_NUM_UNITS, _LQ - _MAX_REL)),
      ],
      axis=1,
  )
  vtp = jnp.concatenate(
      [vt, jnp.broadcast_to(vt[:, -1:], (_NUM_UNITS, 7))], axis=1
  )
  vt8 = jnp.stack([vtp[:, p : p + _LQ + _LK] for p in range(8)])
  vt8 = vt8.reshape(8 * _TR, 8, _LQ + _LK)

  mesh = plsc.VectorSubcoreMesh(
      core_axis_name="c", subcore_axis_name="s", num_cores=_NC, num_subcores=_NS
  )
  out5 = pl.kernel(
      _band_expand,
      out_type=jax.ShapeDtypeStruct((_LQ, _TR, _NTC, 8, 128), jnp.float32),
      mesh=mesh,
      scratch_types=[
          pltpu.VMEM((_TRH, _STC, 8, 128), jnp.float32),
          pltpu.SemaphoreType.DMA,
          pltpu.SemaphoreType.DMA,
      ],
      compiler_params=pltpu.CompilerParams(use_tc_tiling_on_sc=False),
  )(vt8)
  return jnp.transpose(out5, (0, 2, 4, 1, 3)).reshape(_LQ, _LK, _NUM_UNITS)
